# baseline (speedup 1.0000x reference)
import functools

import jax
import jax.numpy as jnp
from jax import lax
from jax.experimental import pallas as pl
from jax.experimental.pallas import tpu as pltpu
from jax.experimental.pallas import tpu_sc as plsc

B = 16384
D = 32

_info = plsc.get_sparse_core_info()
_NC = _info.num_cores
_NS = _info.num_subcores
_NW = _NC * _NS
_BPW = B // _NW
_CHUNK = 128
_NCHUNK = _BPW // _CHUNK

_mesh = plsc.VectorSubcoreMesh(core_axis_name="c", subcore_axis_name="s")


@functools.partial(
    pl.kernel,
    mesh=_mesh,
    out_type=jax.ShapeDtypeStruct((D, B), jnp.float32),
    scratch_types=[
        pltpu.VMEM((_BPW,), jnp.int32),
        pltpu.VMEM((_BPW, D), jnp.float32),
        pltpu.VMEM((D, _BPW), jnp.float32),
        pltpu.SemaphoreType.DMA,
    ],
    compiler_params=pltpu.CompilerParams(
        use_tc_tiling_on_sc=False, needs_layout_passes=False
    ),
)
def _gather_kernel(table_hbm, idx_hbm, outT_hbm, idx_v, rows_v, outT_v, sem):
    wid = lax.axis_index("s") * _NC + lax.axis_index("c")
    base = wid * _BPW
    pltpu.sync_copy(idx_hbm.at[pl.ds(base, _BPW)], idx_v)
    copies = [
        pltpu.async_copy(
            table_hbm.at[idx_v.at[pl.ds(j * _CHUNK, _CHUNK)]],
            rows_v.at[pl.ds(j * _CHUNK, _CHUNK)],
            sem,
        )
        for j in range(_NCHUNK)
    ]
    for c in copies:
        c.wait()

    iota = lax.iota(jnp.int32, 16)

    def body(g, carry):
        rid = g * 16 + iota
        for j in range(D):
            cid = jnp.full((16,), j, jnp.int32)
            v = plsc.load_gather(rows_v, [rid, cid])
            outT_v[j, pl.ds(g * 16, 16)] = v
        return carry

    lax.fori_loop(0, _BPW // 16, body, jnp.int32(0), unroll=False)
    pltpu.sync_copy(outT_v, outT_hbm.at[:, pl.ds(base, _BPW)])


def kernel(broadcaster, table):
    idx = broadcaster.astype(jnp.int32)
    outT = _gather_kernel(table, idx)
    return outT.T


# speedup vs baseline: 1.3328x; 1.3328x over previous
import functools

import jax
import jax.numpy as jnp
from jax import lax
from jax.experimental import pallas as pl
from jax.experimental.pallas import tpu as pltpu
from jax.experimental.pallas import tpu_sc as plsc

B = 16384
D = 32

_info = plsc.get_sparse_core_info()
_NC = _info.num_cores
_NS = _info.num_subcores
_NW = _NC * _NS
_BPW = B // _NW

_mesh = plsc.VectorSubcoreMesh(core_axis_name="c", subcore_axis_name="s")


@functools.partial(
    pl.kernel,
    mesh=_mesh,
    out_type=jax.ShapeDtypeStruct((D, B), jnp.float32),
    scratch_types=[
        pltpu.VMEM((_BPW,), jnp.int32),
        pltpu.VMEM((_BPW, D), jnp.float32),
        pltpu.VMEM((D, _BPW), jnp.float32),
        pltpu.SemaphoreType.DMA,
    ],
    compiler_params=pltpu.CompilerParams(needs_layout_passes=False),
)
def _gather_kernel(table_hbm, idx_hbm, outT_hbm, idx_v, rows_v, outT_v, sem):
    wid = lax.axis_index("s") * _NC + lax.axis_index("c")
    base = wid * _BPW
    pltpu.sync_copy(idx_hbm.at[pl.ds(base, _BPW)], idx_v)

    def issue(g, carry):
        vec = idx_v[pl.ds(g * 16, 16)]
        for k in range(16):
            i = vec[k]
            pltpu.async_copy(table_hbm.at[i], rows_v.at[g * 16 + k], sem)
        return carry

    lax.fori_loop(0, _BPW // 16, issue, jnp.int32(0), unroll=False)

    def drain(p, carry):
        pltpu.make_async_copy(table_hbm.at[0], rows_v.at[p], sem).wait()
        return carry

    lax.fori_loop(0, _BPW, drain, jnp.int32(0), unroll=False)

    iota = lax.iota(jnp.int32, 16)

    def body(g, carry):
        rid = g * 16 + iota
        for j in range(D):
            cid = jnp.full((16,), j, jnp.int32)
            v = plsc.load_gather(rows_v, [rid, cid])
            outT_v[j, pl.ds(g * 16, 16)] = v
        return carry

    lax.fori_loop(0, _BPW // 16, body, jnp.int32(0), unroll=False)
    pltpu.sync_copy(outT_v, outT_hbm.at[:, pl.ds(base, _BPW)])


def kernel(broadcaster, table):
    idx = broadcaster.astype(jnp.int32)
    outT = _gather_kernel(table, idx)
    return outT.T
